# Initial kernel scaffold; baseline (speedup 1.0000x reference)
import functools

import jax
import jax.numpy as jnp
from jax import lax
from jax.experimental import pallas as pl
from jax.experimental.pallas import tpu as pltpu
from jax.experimental.pallas import tpu_sc as plsc

IN_F = 4096
OUT_F = 4096
T = 32
L = 16
NC = 2
NS = 16
NW = NC * NS
B = 128
ROWS_PER_TILE = OUT_F // NS


def _comp_body(nb, chunk, xT_hbm, rows_hbm, cols_hbm, vals_hbm, out_hbm,
               stage_v, cols_v, vals_v, rows_v, gath_v, scaled_v,
               xT_sh, acc_sh):
    c = lax.axis_index("c")
    s = lax.axis_index("s")
    wid = c * NS + s

    def _zero(i, _):
        stage_v[i, pl.ds(0, L)] = jnp.zeros((L,), jnp.float32)
        stage_v[i, pl.ds(L, L)] = jnp.zeros((L,), jnp.float32)
        return 0
    lax.fori_loop(0, ROWS_PER_TILE, _zero, 0)
    pltpu.sync_copy(stage_v, acc_sh.at[pl.ds(s * ROWS_PER_TILE, ROWS_PER_TILE)])

    pltpu.sync_copy(xT_hbm.at[pl.ds(s * (IN_F // NS), IN_F // NS)], stage_v)
    pltpu.sync_copy(stage_v, xT_sh.at[pl.ds(s * (IN_F // NS), IN_F // NS)])

    base = wid * chunk
    pltpu.sync_copy(cols_hbm.at[pl.ds(base, chunk)], cols_v)
    pltpu.sync_copy(vals_hbm.at[pl.ds(base, chunk)], vals_v)
    pltpu.sync_copy(rows_hbm.at[pl.ds(wid * nb, nb)], rows_v)

    plsc.subcore_barrier()

    def _batch(bi, _):
        pltpu.sync_copy(xT_sh.at[cols_v.at[pl.ds(bi * B, B)]], gath_v)

        def _scale(j, _):
            v = vals_v[bi * B + j]
            scaled_v[j, pl.ds(0, L)] = gath_v[j, pl.ds(0, L)] * v
            scaled_v[j, pl.ds(L, L)] = gath_v[j, pl.ds(L, L)] * v
            return 0
        lax.fori_loop(0, B, _scale, 0, unroll=4)

        pltpu.sync_copy(scaled_v, acc_sh.at[rows_v.at[bi]], add=True)
        return 0
    lax.fori_loop(0, nb, _batch, 0)

    plsc.subcore_barrier()

    pltpu.sync_copy(acc_sh.at[pl.ds(s * ROWS_PER_TILE, ROWS_PER_TILE)], stage_v)
    pltpu.sync_copy(stage_v, out_hbm.at[c, pl.ds(s * ROWS_PER_TILE, ROWS_PER_TILE)])


def _sc_comp(xT, rows2d, cols, vals, nb, chunk):
    mesh = plsc.VectorSubcoreMesh(core_axis_name="c", subcore_axis_name="s")
    body = functools.partial(_comp_body, nb, chunk)
    return pl.kernel(
        body,
        out_type=jax.ShapeDtypeStruct((NC, OUT_F, T), jnp.float32),
        mesh=mesh,
        scratch_types=[
            pltpu.VMEM((ROWS_PER_TILE, T), jnp.float32),
            pltpu.VMEM((chunk,), jnp.int32),
            pltpu.VMEM((chunk,), jnp.float32),
            pltpu.VMEM((nb, B), jnp.int32),
            pltpu.VMEM((B, T), jnp.float32),
            pltpu.VMEM((B, T), jnp.float32),
            pltpu.VMEM_SHARED((IN_F, T), jnp.float32),
            pltpu.VMEM_SHARED((OUT_F, T), jnp.float32),
        ],
    )(xT, rows2d, cols, vals)


def _dense_body(x_ref, w_ref, b_ref, o_ref):
    o_ref[...] = lax.dot_general(
        x_ref[...], w_ref[...], (((1,), (1,)), ((), ())),
        preferred_element_type=jnp.float32) + b_ref[...]


def _tc_dense(x, W, b2d):
    blk = 512
    return pl.pallas_call(
        _dense_body,
        grid=(OUT_F // blk,),
        in_specs=[
            pl.BlockSpec((T, IN_F), lambda i: (0, 0)),
            pl.BlockSpec((blk, IN_F), lambda i: (i, 0)),
            pl.BlockSpec((1, blk), lambda i: (0, i)),
        ],
        out_specs=pl.BlockSpec((T, blk), lambda i: (0, i)),
        out_shape=jax.ShapeDtypeStruct((T, OUT_F), jnp.float32),
    )(x, W, b2d)


def kernel(x, W, b, delta_rows, delta_cols, delta_vals):
    nnz = delta_vals.shape[0]
    chunk = -(-nnz // (NW * B)) * B
    nb = chunk // B
    nnz_pad = NW * chunk
    pad = nnz_pad - nnz
    rows_p = jnp.pad(delta_rows, (0, pad))
    cols_p = jnp.pad(delta_cols, (0, pad))
    vals_p = jnp.pad(delta_vals, (0, pad))

    xT = x.astype(jnp.float32).T
    comp = _sc_comp(xT, rows_p.reshape(NW * nb, B), cols_p, vals_p, nb, chunk)
    dense = _tc_dense(x, W, b.reshape(1, OUT_F))
    return dense + (comp[0] + comp[1]).T


# speedup vs baseline: 12.8953x; 12.8953x over previous
import functools

import jax
import jax.numpy as jnp
from jax import lax
from jax.experimental import pallas as pl
from jax.experimental.pallas import tpu as pltpu
from jax.experimental.pallas import tpu_sc as plsc

IN_F = 4096
OUT_F = 4096
T = 32
L = 16
NC = 2
NS = 16
NW = NC * NS
B = 128
ROWS_PER_TILE = OUT_F // NS


def _comp_body(nb, chunk, xT_hbm, rows_hbm, cols_hbm, vals_hbm, out_hbm,
               stage_v, cols_v, vals_v, rows_v, gath_v, scaled_v,
               bcols_v, brows_v, xT_sh, acc_sh):
    c = lax.axis_index("c")
    s = lax.axis_index("s")
    wid = c * NS + s

    def _zero(i, _):
        stage_v[i, pl.ds(0, L)] = jnp.zeros((L,), jnp.float32)
        stage_v[i, pl.ds(L, L)] = jnp.zeros((L,), jnp.float32)
        return 0
    lax.fori_loop(0, ROWS_PER_TILE, _zero, 0)
    pltpu.sync_copy(stage_v, acc_sh.at[pl.ds(s * ROWS_PER_TILE, ROWS_PER_TILE)])

    pltpu.sync_copy(xT_hbm.at[pl.ds(s * (IN_F // NS), IN_F // NS)], stage_v)
    pltpu.sync_copy(stage_v, xT_sh.at[pl.ds(s * (IN_F // NS), IN_F // NS)])

    base = wid * chunk
    pltpu.sync_copy(cols_hbm.at[pl.ds(base, chunk)], cols_v)
    pltpu.sync_copy(vals_hbm.at[pl.ds(base, chunk)], vals_v)
    pltpu.sync_copy(rows_hbm.at[wid], rows_v)

    plsc.subcore_barrier()

    def _batch(bi, _):
        for k in range(B // L):
            bcols_v[pl.ds(k * L, L)] = cols_v[pl.ds(bi * B + k * L, L)]
            brows_v[pl.ds(k * L, L)] = rows_v[bi, pl.ds(k * L, L)]

        pltpu.sync_copy(xT_sh.at[bcols_v], gath_v)

        def _scale(k, _):
            vv = vals_v[pl.ds(bi * B + k * L, L)]
            for j in range(L):
                r = k * L + j
                v = vv[j]
                scaled_v[r, pl.ds(0, L)] = gath_v[r, pl.ds(0, L)] * v
                scaled_v[r, pl.ds(L, L)] = gath_v[r, pl.ds(L, L)] * v
            return 0
        lax.fori_loop(0, B // L, _scale, 0)

        pltpu.sync_copy(scaled_v, acc_sh.at[brows_v], add=True)
        return 0
    lax.fori_loop(0, nb, _batch, 0)

    plsc.subcore_barrier()

    pltpu.sync_copy(acc_sh.at[pl.ds(s * ROWS_PER_TILE, ROWS_PER_TILE)], stage_v)
    pltpu.sync_copy(stage_v, out_hbm.at[c, pl.ds(s * ROWS_PER_TILE, ROWS_PER_TILE)])


def _sc_comp(xT, rows2d, cols, vals, nb, chunk):
    mesh = plsc.VectorSubcoreMesh(core_axis_name="c", subcore_axis_name="s")
    body = functools.partial(_comp_body, nb, chunk)
    return pl.kernel(
        body,
        out_type=jax.ShapeDtypeStruct((NC, OUT_F, T), jnp.float32),
        mesh=mesh,
        compiler_params=pltpu.CompilerParams(use_tc_tiling_on_sc=False),
        scratch_types=[
            pltpu.VMEM((ROWS_PER_TILE, T), jnp.float32),
            pltpu.VMEM((chunk,), jnp.int32),
            pltpu.VMEM((chunk,), jnp.float32),
            pltpu.VMEM((nb, B), jnp.int32),
            pltpu.VMEM((B, T), jnp.float32),
            pltpu.VMEM((B, T), jnp.float32),
            pltpu.VMEM((B,), jnp.int32),
            pltpu.VMEM((B,), jnp.int32),
            pltpu.VMEM_SHARED((IN_F, T), jnp.float32),
            pltpu.VMEM_SHARED((OUT_F, T), jnp.float32),
        ],
    )(xT, rows2d, cols, vals)


def _dense_body(x_ref, w_ref, b_ref, o_ref):
    o_ref[...] = lax.dot_general(
        x_ref[...], w_ref[...], (((1,), (1,)), ((), ())),
        preferred_element_type=jnp.float32,
        precision=lax.Precision.HIGHEST) + b_ref[...]


def _tc_dense(x, W, b2d):
    blk = 512
    return pl.pallas_call(
        _dense_body,
        grid=(OUT_F // blk,),
        in_specs=[
            pl.BlockSpec((T, IN_F), lambda i: (0, 0)),
            pl.BlockSpec((blk, IN_F), lambda i: (i, 0)),
            pl.BlockSpec((1, blk), lambda i: (0, i)),
        ],
        out_specs=pl.BlockSpec((T, blk), lambda i: (0, i)),
        out_shape=jax.ShapeDtypeStruct((T, OUT_F), jnp.float32),
    )(x, W, b2d)


def kernel(x, W, b, delta_rows, delta_cols, delta_vals):
    nnz = delta_vals.shape[0]
    chunk = -(-nnz // (NW * B)) * B
    nb = chunk // B
    nnz_pad = NW * chunk
    pad = nnz_pad - nnz
    rows_p = jnp.pad(delta_rows, (0, pad))
    cols_p = jnp.pad(delta_cols, (0, pad))
    vals_p = jnp.pad(delta_vals, (0, pad))

    xT = x.astype(jnp.float32).T
    comp = _sc_comp(xT, rows_p.reshape(NW, nb, B), cols_p, vals_p, nb, chunk)
    dense = _tc_dense(x, W, b.reshape(1, OUT_F))
    return dense + (comp[0] + comp[1]).T
